# baseline (speedup 1.0000x reference)
import functools

import jax
import jax.numpy as jnp
from jax.experimental import pallas as pl
from jax.experimental.pallas import tpu as pltpu

_PADL = 128


def _conv_pool_kernel(x_ref, w_ref, maxp_ref, minp_ref, stats_ref,
                      xpad_ref, acc_ref, *, KH, KW, H, W):
    Cout = maxp_ref.shape[2]
    Cin = x_ref.shape[1]
    HW = H * W
    Ho2, Wo2 = H // 2, W // 2

    xpad_ref[:, pl.ds(0, _PADL)] = jnp.zeros((Cin, _PADL), jnp.bfloat16)
    xpad_ref[:, pl.ds(_PADL + HW, _PADL)] = jnp.zeros((Cin, _PADL),
                                                      jnp.bfloat16)
    xpad_ref[:, pl.ds(_PADL, HW)] = x_ref[0].astype(jnp.bfloat16)

    wpos = jax.lax.broadcasted_iota(jnp.int32, (1, HW), 1) % W
    not_first = wpos > 0
    not_last = wpos < (W - 1)

    pieces = []
    for kh in range(KH):
        for kw in range(KW):
            s = (kh - KH // 2) * W + (kw - KW // 2)
            piece = xpad_ref[:, pl.ds(_PADL + s, HW)]
            if kw == 0:
                piece = jnp.where(not_first, piece, jnp.bfloat16(0))
            elif kw == KW - 1:
                piece = jnp.where(not_last, piece, jnp.bfloat16(0))
            pieces.append(piece)
    xcm = jnp.concatenate(pieces, axis=0)

    acc = jax.lax.dot_general(
        xcm, w_ref[...], (((0,), (0,)), ((), ())),
        preferred_element_type=jnp.float32)

    ssum = jnp.sum(acc, axis=0, keepdims=True)
    ssq = jnp.sum(acc * acc, axis=0, keepdims=True)
    stats_ref[...] = jnp.concatenate([ssum, ssq], axis=0).reshape(1, 2, Cout)

    acc_ref[...] = acc.reshape(H, W, Cout)
    mx = None
    mn = None
    for di in range(2):
        for dj in range(2):
            part = acc_ref[pl.ds(di, Ho2, 2), pl.ds(dj, Wo2, 2), :]
            mx = part if mx is None else jnp.maximum(mx, part)
            mn = part if mn is None else jnp.minimum(mn, part)
    maxp_ref[...] = mx.reshape(1, Ho2 * Wo2, Cout).astype(maxp_ref.dtype)
    minp_ref[...] = mn.reshape(1, Ho2 * Wo2, Cout).astype(minp_ref.dtype)


def _bn_relu_kernel(stats_ref, g_ref, be_ref, maxp_ref, minp_ref, o_ref,
                    *, count, eps):
    _, Cout, P = o_ref.shape

    ssum = jnp.sum(stats_ref[:, 0, :], axis=0, keepdims=True)
    ssq = jnp.sum(stats_ref[:, 1, :], axis=0, keepdims=True)
    mean = ssum / count
    var = ssq / count - mean * mean
    inv = jax.lax.rsqrt(var + eps)
    scale = g_ref[...] * inv
    shift = be_ref[...] - mean * scale

    sel = jnp.where(scale >= 0.0,
                    maxp_ref[0].astype(jnp.float32),
                    minp_ref[0].astype(jnp.float32))
    y = jnp.maximum(sel * scale + shift, 0.0)

    o_ref[...] = y.T.reshape(1, Cout, P)


def kernel(x, w, b, gamma, beta):
    del b
    eps = 1e-5
    N, Cin, H, W = x.shape
    Cout, _, KH, KW = w.shape
    assert H % 2 == 0 and W % 2 == 0
    Ho2, Wo2 = H // 2, W // 2
    HW = H * W
    P = Ho2 * Wo2

    x3 = x.reshape(N, Cin, HW)
    wmat = jnp.transpose(w, (2, 3, 1, 0)).astype(jnp.bfloat16)
    wmat = wmat.reshape(KH * KW * Cin, Cout)
    g2 = gamma.reshape(1, Cout).astype(jnp.float32)
    be2 = beta.reshape(1, Cout).astype(jnp.float32)

    k1 = functools.partial(_conv_pool_kernel, KH=KH, KW=KW, H=H, W=W)
    flops1 = 2 * N * HW * KH * KW * Cin * Cout
    bytes1 = (4 * x3.size + 2 * wmat.size
              + 2 * 2 * N * P * Cout + 4 * 2 * N * Cout)
    maxp, minp, stats = pl.pallas_call(
        k1,
        grid=(N,),
        in_specs=[
            pl.BlockSpec((1, Cin, HW), lambda n: (n, 0, 0)),
            pl.BlockSpec((KH * KW * Cin, Cout), lambda n: (0, 0)),
        ],
        out_specs=[
            pl.BlockSpec((1, P, Cout), lambda n: (n, 0, 0)),
            pl.BlockSpec((1, P, Cout), lambda n: (n, 0, 0)),
            pl.BlockSpec((1, 2, Cout), lambda n: (n, 0, 0)),
        ],
        out_shape=[
            jax.ShapeDtypeStruct((N, P, Cout), jnp.bfloat16),
            jax.ShapeDtypeStruct((N, P, Cout), jnp.bfloat16),
            jax.ShapeDtypeStruct((N, 2, Cout), jnp.float32),
        ],
        scratch_shapes=[
            pltpu.VMEM((Cin, _PADL + HW + _PADL), jnp.bfloat16),
            pltpu.VMEM((H, W, Cout), jnp.float32),
        ],
        compiler_params=pltpu.CompilerParams(dimension_semantics=("parallel",)),
        cost_estimate=pl.CostEstimate(flops=flops1, transcendentals=0,
                                      bytes_accessed=bytes1),
    )(x3, wmat)

    count = N * HW
    k2 = functools.partial(_bn_relu_kernel, count=count, eps=eps)
    flops2 = 8 * N * P * Cout
    bytes2 = (4 * stats.size + 4 * 2 * Cout
              + 2 * 2 * N * P * Cout + 4 * N * P * Cout)
    out = pl.pallas_call(
        k2,
        grid=(N,),
        in_specs=[
            pl.BlockSpec((N, 2, Cout), lambda n: (0, 0, 0)),
            pl.BlockSpec((1, Cout), lambda n: (0, 0)),
            pl.BlockSpec((1, Cout), lambda n: (0, 0)),
            pl.BlockSpec((1, P, Cout), lambda n: (n, 0, 0)),
            pl.BlockSpec((1, P, Cout), lambda n: (n, 0, 0)),
        ],
        out_specs=pl.BlockSpec((1, Cout, P), lambda n: (n, 0, 0)),
        out_shape=jax.ShapeDtypeStruct((N, Cout, P), jnp.float32),
        compiler_params=pltpu.CompilerParams(dimension_semantics=("parallel",)),
        cost_estimate=pl.CostEstimate(flops=flops2, transcendentals=0,
                                      bytes_accessed=bytes2),
    )(stats, g2, be2, maxp, minp)

    return out.reshape(N, Cout, Ho2, Wo2)


# speedup vs baseline: 1.2674x; 1.2674x over previous
import functools

import jax
import jax.numpy as jnp
from jax.experimental import pallas as pl
from jax.experimental.pallas import tpu as pltpu


def _conv_pool_kernel(x_ref, w_ref, maxp_ref, minp_ref, stats_ref,
                      xpad_ref, acc_ref, *, KH, KW, H, W):
    Cout = maxp_ref.shape[2]
    Cin = x_ref.shape[1]
    HW = H * W
    Ho2, Wo2 = H // 2, W // 2

    xpad_ref[...] = jnp.zeros_like(xpad_ref)
    xrm = x_ref[0].T
    xpad_ref[pl.ds(1, H), pl.ds(1, W), :] = xrm.reshape(H, W, Cin)

    cols = []
    for kh in range(KH):
        for kw in range(KW):
            cols.append(
                xpad_ref[pl.ds(kh, H), pl.ds(kw, W), :].reshape(HW, Cin))
    patch = jnp.concatenate(cols, axis=1).astype(jnp.bfloat16)

    acc = jnp.dot(patch, w_ref[...],
                  preferred_element_type=jnp.float32)

    ssum = jnp.sum(acc, axis=0, keepdims=True)
    ssq = jnp.sum(acc * acc, axis=0, keepdims=True)
    stats_ref[...] = jnp.concatenate([ssum, ssq], axis=0).reshape(1, 2, Cout)

    acc_ref[...] = acc.reshape(H, W, Cout)
    mx = None
    mn = None
    for di in range(2):
        for dj in range(2):
            part = acc_ref[pl.ds(di, Ho2, 2), pl.ds(dj, Wo2, 2), :]
            mx = part if mx is None else jnp.maximum(mx, part)
            mn = part if mn is None else jnp.minimum(mn, part)
    maxp_ref[...] = mx.reshape(1, Ho2 * Wo2, Cout).astype(maxp_ref.dtype)
    minp_ref[...] = mn.reshape(1, Ho2 * Wo2, Cout).astype(minp_ref.dtype)


def _bn_relu_kernel(stats_ref, g_ref, be_ref, maxp_ref, minp_ref, o_ref,
                    *, count, eps):
    _, Cout, P = o_ref.shape

    ssum = jnp.sum(stats_ref[:, 0, :], axis=0, keepdims=True)
    ssq = jnp.sum(stats_ref[:, 1, :], axis=0, keepdims=True)
    mean = ssum / count
    var = ssq / count - mean * mean
    inv = jax.lax.rsqrt(var + eps)
    scale = g_ref[...] * inv
    shift = be_ref[...] - mean * scale

    sel = jnp.where(scale >= 0.0,
                    maxp_ref[0].astype(jnp.float32),
                    minp_ref[0].astype(jnp.float32))
    y = jnp.maximum(sel * scale + shift, 0.0)

    o_ref[...] = y.T.reshape(1, Cout, P)


def kernel(x, w, b, gamma, beta):
    del b
    eps = 1e-5
    N, Cin, H, W = x.shape
    Cout, _, KH, KW = w.shape
    assert H % 2 == 0 and W % 2 == 0
    Ho2, Wo2 = H // 2, W // 2
    HW = H * W
    P = Ho2 * Wo2

    x3 = x.reshape(N, Cin, HW)
    wmat = jnp.transpose(w, (2, 3, 1, 0)).astype(jnp.bfloat16)
    wmat = wmat.reshape(KH * KW * Cin, Cout)
    g2 = gamma.reshape(1, Cout).astype(jnp.float32)
    be2 = beta.reshape(1, Cout).astype(jnp.float32)

    k1 = functools.partial(_conv_pool_kernel, KH=KH, KW=KW, H=H, W=W)
    flops1 = 2 * N * HW * KH * KW * Cin * Cout
    bytes1 = (4 * x3.size + 2 * wmat.size
              + 2 * 2 * N * P * Cout + 4 * 2 * N * Cout)
    maxp, minp, stats = pl.pallas_call(
        k1,
        grid=(N,),
        in_specs=[
            pl.BlockSpec((1, Cin, HW), lambda n: (n, 0, 0)),
            pl.BlockSpec((KH * KW * Cin, Cout), lambda n: (0, 0)),
        ],
        out_specs=[
            pl.BlockSpec((1, P, Cout), lambda n: (n, 0, 0)),
            pl.BlockSpec((1, P, Cout), lambda n: (n, 0, 0)),
            pl.BlockSpec((1, 2, Cout), lambda n: (n, 0, 0)),
        ],
        out_shape=[
            jax.ShapeDtypeStruct((N, P, Cout), jnp.bfloat16),
            jax.ShapeDtypeStruct((N, P, Cout), jnp.bfloat16),
            jax.ShapeDtypeStruct((N, 2, Cout), jnp.float32),
        ],
        scratch_shapes=[
            pltpu.VMEM((H + 2, W + 2, Cin), jnp.float32),
            pltpu.VMEM((H, W, Cout), jnp.float32),
        ],
        compiler_params=pltpu.CompilerParams(dimension_semantics=("parallel",)),
        cost_estimate=pl.CostEstimate(flops=flops1, transcendentals=0,
                                      bytes_accessed=bytes1),
    )(x3, wmat)

    count = N * HW
    k2 = functools.partial(_bn_relu_kernel, count=count, eps=eps)
    flops2 = 8 * N * P * Cout
    bytes2 = (4 * stats.size + 4 * 2 * Cout
              + 2 * 2 * N * P * Cout + 4 * N * P * Cout)
    out = pl.pallas_call(
        k2,
        grid=(N,),
        in_specs=[
            pl.BlockSpec((N, 2, Cout), lambda n: (0, 0, 0)),
            pl.BlockSpec((1, Cout), lambda n: (0, 0)),
            pl.BlockSpec((1, Cout), lambda n: (0, 0)),
            pl.BlockSpec((1, P, Cout), lambda n: (n, 0, 0)),
            pl.BlockSpec((1, P, Cout), lambda n: (n, 0, 0)),
        ],
        out_specs=pl.BlockSpec((1, Cout, P), lambda n: (n, 0, 0)),
        out_shape=jax.ShapeDtypeStruct((N, Cout, P), jnp.float32),
        compiler_params=pltpu.CompilerParams(dimension_semantics=("parallel",)),
        cost_estimate=pl.CostEstimate(flops=flops2, transcendentals=0,
                                      bytes_accessed=bytes2),
    )(stats, g2, be2, maxp, minp)

    return out.reshape(N, Cout, Ho2, Wo2)
